# Initial kernel scaffold; baseline (speedup 1.0000x reference)
import jax
import jax.numpy as jnp
from jax.experimental import pallas as pl


def kernel(x, edge_index, W1, b1, W2, b2, W3, b3):
    raise NotImplementedError("write your pallas kernel here")


# speedup vs baseline: 19.3253x; 19.3253x over previous
import functools

import jax
import jax.numpy as jnp
from jax import lax
from jax.experimental import pallas as pl
from jax.experimental.pallas import tpu as pltpu
from jax.experimental.pallas import tpu_sc as plsc

N_TILES = 32
N_SUB = 16
BATCH = 128


def _edge_agg_kernel(n_pad, width, nb, gather):
    rpt = n_pad // N_SUB
    mesh = plsc.VectorSubcoreMesh(core_axis_name="c", subcore_axis_name="s")

    scratch = [
        pltpu.VMEM((nb, BATCH), jnp.int32),
        pltpu.VMEM((BATCH, width), jnp.float32),
        pltpu.VMEM_SHARED((n_pad, width), jnp.float32),
        pltpu.SemaphoreType.DMA,
    ]
    if gather:
        scratch.insert(0, pltpu.VMEM((nb, BATCH), jnp.int32))

    @functools.partial(
        pl.kernel,
        mesh=mesh,
        out_type=jax.ShapeDtypeStruct((2, n_pad, width), jnp.float32),
        scratch_types=scratch,
        compiler_params=pltpu.CompilerParams(use_tc_tiling_on_sc=False),
    )
    def k(*refs):
        if gather:
            (hh_hbm, src_hbm, dst_hbm, zero_hbm, out_hbm,
             src_v, dst_v, rows_v, acc_sh, sem) = refs
        else:
            (ones_hbm, dst_hbm, zero_hbm, out_hbm,
             dst_v, rows_v, acc_sh, sem) = refs
        c = lax.axis_index("c")
        s = lax.axis_index("s")
        w = s * 2 + c

        if gather:
            pltpu.sync_copy(src_hbm.at[w], src_v)
        else:
            pltpu.sync_copy(ones_hbm, rows_v)
        pltpu.sync_copy(dst_hbm.at[w], dst_v)
        pltpu.sync_copy(zero_hbm, acc_sh.at[pl.ds(s * rpt, rpt)])
        plsc.subcore_barrier()

        def body(j, carry):
            if gather:
                pltpu.async_copy(hh_hbm.at[src_v.at[j]], rows_v, sem).wait()
            pltpu.sync_copy(rows_v, acc_sh.at[dst_v.at[j]], add=True)
            return carry

        lax.fori_loop(0, nb, body, 0)
        plsc.subcore_barrier()
        pltpu.sync_copy(acc_sh.at[pl.ds(s * rpt, rpt)],
                        out_hbm.at[c, pl.ds(s * rpt, rpt)])

    return k


def _tc_first(n_pad, n, d_in, h1):
    def body(x_ref, w_ref, degp_ref, hh_ref, dinv_ref):
        deg = degp_ref[0, :, 0:1] + degp_ref[1, :, 0:1] + 1.0
        dinv = lax.rsqrt(deg)
        h = jnp.dot(x_ref[...], w_ref[...],
                    preferred_element_type=jnp.float32,
                    precision=lax.Precision.HIGHEST)
        hh_ref[...] = h * dinv
        dinv_ref[...] = jnp.broadcast_to(dinv, (n_pad, 16))

    return pl.pallas_call(
        body,
        out_shape=[
            jax.ShapeDtypeStruct((n_pad, h1), jnp.float32),
            jax.ShapeDtypeStruct((n_pad, 16), jnp.float32),
        ],
    )


def _tc_mid(n_pad, n, w_in, w_out):
    def body(p_ref, hh_ref, dinv_ref, b_ref, w_ref, out_ref):
        dinv = dinv_ref[:, 0:1]
        z = dinv * (p_ref[0] + p_ref[1] + hh_ref[...]) + b_ref[...]
        xn = jnp.maximum(z, 0.0)
        h = jnp.dot(xn, w_ref[...],
                    preferred_element_type=jnp.float32,
                    precision=lax.Precision.HIGHEST)
        valid = lax.broadcasted_iota(jnp.int32, (n_pad, 1), 0) < n
        out_ref[...] = jnp.where(valid, h * dinv, 0.0)

    return pl.pallas_call(
        body,
        out_shape=jax.ShapeDtypeStruct((n_pad, w_out), jnp.float32),
    )


def _tc_last(n_pad, w_out):
    def body(p_ref, hh_ref, dinv_ref, b_ref, out_ref):
        dinv = dinv_ref[:, 0:1]
        z = dinv * (p_ref[0] + p_ref[1] + hh_ref[...]) + b_ref[...]
        m = jnp.max(z, axis=1, keepdims=True)
        lse = jnp.log(jnp.sum(jnp.exp(z - m), axis=1, keepdims=True)) + m
        out_ref[...] = z - lse

    return pl.pallas_call(
        body,
        out_shape=jax.ShapeDtypeStruct((n_pad, w_out), jnp.float32),
    )


def kernel(x, edge_index, W1, b1, W2, b2, W3, b3):
    n, d_in = x.shape
    e = edge_index.shape[1]
    h1 = W1.shape[1]
    h2 = W2.shape[1]
    d_out = W3.shape[1]

    n_pad = (n + N_SUB + 127) // 128 * 128
    nb = (e + N_TILES * BATCH - 1) // (N_TILES * BATCH)
    e_pad = N_TILES * nb * BATCH
    rpt = n_pad // N_SUB

    src = edge_index[0].astype(jnp.int32)
    dst = edge_index[1].astype(jnp.int32)
    pad = jnp.full((e_pad - e,), n, jnp.int32)
    src3 = jnp.concatenate([src, pad]).reshape(N_TILES, nb, BATCH)
    dst3 = jnp.concatenate([dst, pad]).reshape(N_TILES, nb, BATCH)

    x_pad = jnp.pad(x, ((0, n_pad - n), (0, 0)))
    zero64 = jnp.zeros((rpt, h1), jnp.float32)
    zero16 = jnp.zeros((rpt, 16), jnp.float32)
    ones_col = jnp.zeros((BATCH, 16), jnp.float32).at[:, 0].set(1.0)

    deg_k = _edge_agg_kernel(n_pad, 16, nb, gather=False)
    agg64 = _edge_agg_kernel(n_pad, h1, nb, gather=True)
    agg16 = _edge_agg_kernel(n_pad, d_out, nb, gather=True)

    degp = deg_k(ones_col, dst3, zero16)
    hh1, dinv16 = _tc_first(n_pad, n, d_in, h1)(x_pad, W1, degp)
    p1 = agg64(hh1, src3, dst3, zero64)
    hh2 = _tc_mid(n_pad, n, h1, h2)(p1, hh1, dinv16, b1.reshape(1, -1), W2)
    p2 = agg64(hh2, src3, dst3, zero64)
    hh3 = _tc_mid(n_pad, n, h2, d_out)(p2, hh2, dinv16, b2.reshape(1, -1), W3)
    p3 = agg16(hh3, src3, dst3, zero16)
    out = _tc_last(n_pad, d_out)(p3, hh3, dinv16, b3.reshape(1, -1))
    return out[:n]
